# baseline (speedup 1.0000x reference)
import functools

import jax
import jax.numpy as jnp
from jax import lax
from jax.experimental import pallas as pl
from jax.experimental.pallas import tpu as pltpu
from jax.experimental.pallas import tpu_sc as plsc

F32 = jnp.float32
I32 = jnp.int32
VP = 128
B = 256
BN = 1000
NW = 32


def _make_cnt(N, E, R, V):
    KC = 128
    RC = 16
    EW = E // NW
    NCH = EW // KC
    assert EW == NCH * KC + RC
    ZR = 80
    NU = N // ZR
    mesh = plsc.VectorSubcoreMesh(core_axis_name="c", subcore_axis_name="s")

    @functools.partial(
        pl.kernel, mesh=mesh,
        out_type=(jax.ShapeDtypeStruct((2, N, 128), F32),
                  jax.ShapeDtypeStruct((E,), I32)),
        scratch_types=[
            pltpu.VMEM((KC,), I32),
            pltpu.VMEM((KC,), I32),
            pltpu.VMEM((KC,), I32),
            pltpu.VMEM((KC,), I32),
            pltpu.VMEM((RC,), I32),
            pltpu.VMEM((RC,), I32),
            pltpu.VMEM((RC,), I32),
            pltpu.VMEM((RC,), I32),
            pltpu.VMEM((KC, 128), F32),
            pltpu.VMEM((ZR, 128), F32),
            pltpu.VMEM_SHARED((N, 128), F32),
        ],
    )
    def k(src_hbm, dst_hbm, rel_hbm, types_hbm, out_hbm, tsrc_hbm,
          srcb, dstb, relb, tsrcb, srcb2, dstb2, relb2, tsrcb2, ohb, zb, acc):
        cid = lax.axis_index("c")
        sid = lax.axis_index("s")
        wid = sid * 2 + cid
        base = wid * EW
        zeros16 = jnp.zeros((16,), F32)
        iota = lax.iota(I32, 16)

        def zrow(i, carry):
            for j in range(8):
                zb[i, pl.ds(j * 16, 16)] = zeros16
            return carry
        lax.fori_loop(0, ZR, zrow, 0)

        def zoh(i, carry):
            for j in range(8):
                ohb[i, pl.ds(j * 16, 16)] = zeros16
            return carry
        lax.fori_loop(0, KC, zoh, 0)
        for t in range(pl.cdiv(NU, 16)):
            u = sid + t * 16

            @pl.when(u < NU)
            def _():
                pltpu.sync_copy(zb, acc.at[pl.ds(u * ZR, ZR)])
        plsc.subcore_barrier()

        def do_chunk(off, sz, sb, db, rb, tb):
            pltpu.sync_copy(src_hbm.at[pl.ds(off, sz)], sb)
            pltpu.sync_copy(dst_hbm.at[pl.ds(off, sz)], db)
            pltpu.sync_copy(rel_hbm.at[pl.ds(off, sz)], rb)
            pltpu.sync_copy(types_hbm.at[sb], tb)
            pltpu.sync_copy(tb, tsrc_hbm.at[pl.ds(off, sz)])

            def grp(j, carry2):
                r16 = rb[pl.ds(j * 16, 16)]
                for kk in range(16):
                    rv = jnp.full((16,), r16[kk], I32)
                    ohb[j * 16 + kk, pl.ds(0, 16)] = (
                        jnp.where(iota == rv, 1.0, 0.0).astype(F32))
                return carry2
            lax.fori_loop(0, sz // 16, grp, 0)
            oslice = ohb.at[pl.ds(0, sz)] if sz != KC else ohb
            pltpu.sync_copy(oslice, acc.at[db], add=True)

        def chunk(c, carry):
            do_chunk(base + c * KC, KC, srcb, dstb, relb, tsrcb)
            return carry
        lax.fori_loop(0, NCH, chunk, 0)
        do_chunk(base + NCH * KC, RC, srcb2, dstb2, relb2, tsrcb2)

        plsc.subcore_barrier()
        for t in range(pl.cdiv(NU, 16)):
            u = sid + t * 16

            @pl.when(u < NU)
            def _():
                pltpu.sync_copy(acc.at[pl.ds(u * ZR, ZR)],
                                out_hbm.at[cid, pl.ds(u * ZR, ZR)])

    return k


def _make_edges(N, E, R, H, layer):
    KC = 128
    RC = 16
    EW = E // NW
    NCH = EW // KC
    assert EW == NCH * KC + RC
    ZR = 80
    NU = N // ZR
    mesh = plsc.VectorSubcoreMesh(core_axis_name="c", subcore_axis_name="s")

    scratch = [
        pltpu.VMEM((KC,), I32),
        pltpu.VMEM((KC,), I32),
        pltpu.VMEM((KC,), F32),
        pltpu.VMEM((RC,), I32),
        pltpu.VMEM((RC,), I32),
        pltpu.VMEM((RC,), F32),
        pltpu.VMEM((KC, H), F32),
        pltpu.VMEM((ZR, H), F32),
        pltpu.VMEM_SHARED((N, H), F32),
    ]
    if layer == 1:
        scratch += [pltpu.VMEM((KC,), I32), pltpu.VMEM((RC,), I32)]
        out_type = (jax.ShapeDtypeStruct((2, N, H), F32),
                    jax.ShapeDtypeStruct((E,), F32))
    else:
        out_type = jax.ShapeDtypeStruct((2, N, H), F32)

    def body(*args):
        if layer == 1:
            (gidx_hbm, dst_hbm, key_hbm, nflat_hbm, table_hbm,
             out_hbm, norme_hbm, g0, dstb, n0, gidxb2, dstb2, nrmb2,
             rows0, zb, acc, keyb, keyb2) = args
        else:
            (gidx_hbm, dst_hbm, norme_hbm, table_hbm,
             out_hbm, g0, dstb, n0, gidxb2, dstb2, nrmb2,
             rows0, zb, acc) = args
        gb_ = (g0,)
        nb_ = (n0,)
        rows_ = (rows0,)
        cid = lax.axis_index("c")
        sid = lax.axis_index("s")
        wid = sid * 2 + cid
        base = wid * EW
        zeros16 = jnp.zeros((16,), F32)

        def zrow(i, carry):
            for j in range(H // 16):
                zb[i, pl.ds(j * 16, 16)] = zeros16
            return carry
        lax.fori_loop(0, ZR, zrow, 0)
        for t in range(pl.cdiv(NU, 16)):
            u = sid + t * 16

            @pl.when(u < NU)
            def _():
                pltpu.sync_copy(zb, acc.at[pl.ds(u * ZR, ZR)])
        plsc.subcore_barrier()

        def do_chunk(off, bi):
            pltpu.sync_copy(gidx_hbm.at[pl.ds(off, KC)], gb_[bi])
            pltpu.sync_copy(dst_hbm.at[pl.ds(off, KC)], dstb)
            if layer == 1:
                pltpu.sync_copy(key_hbm.at[pl.ds(off, KC)], keyb)
                pltpu.sync_copy(nflat_hbm.at[keyb], nb_[bi])
                pltpu.sync_copy(nb_[bi], norme_hbm.at[pl.ds(off, KC)])
            else:
                pltpu.sync_copy(norme_hbm.at[pl.ds(off, KC)], nb_[bi])
            rows = rows_[bi]
            nb = nb_[bi]
            pltpu.sync_copy(table_hbm.at[gb_[bi]], rows)

            def scale(jg, carry2):
                nv16 = nb[pl.ds(jg * 16, 16)]
                for kk in range(16):
                    row = jg * 16 + kk
                    nv = jnp.full((16,), nv16[kk], F32)
                    for j in range(H // 16):
                        rows[row, pl.ds(j * 16, 16)] = (
                            rows[row, pl.ds(j * 16, 16)] * nv)
                return carry2
            lax.fori_loop(0, KC // 16, scale, 0)
            pltpu.sync_copy(rows, acc.at[dstb], add=True)

        def chunk(c, carry):
            do_chunk(base + c * KC, 0)
            return carry
        lax.fori_loop(0, NCH, chunk, 0)

        roff = base + NCH * KC
        pltpu.sync_copy(gidx_hbm.at[pl.ds(roff, RC)], gidxb2)
        pltpu.sync_copy(dst_hbm.at[pl.ds(roff, RC)], dstb2)
        if layer == 1:
            pltpu.sync_copy(key_hbm.at[pl.ds(roff, RC)], keyb2)
            pltpu.sync_copy(nflat_hbm.at[keyb2], nrmb2)
            pltpu.sync_copy(nrmb2, norme_hbm.at[pl.ds(roff, RC)])
        else:
            pltpu.sync_copy(norme_hbm.at[pl.ds(roff, RC)], nrmb2)
        rslice = rows0.at[pl.ds(0, RC)]
        pltpu.sync_copy(table_hbm.at[gidxb2], rslice)

        def scale2(jg, carry2):
            nv16 = nrmb2[pl.ds(jg * 16, 16)]
            for kk in range(16):
                row = jg * 16 + kk
                nv = jnp.full((16,), nv16[kk], F32)
                for j in range(H // 16):
                    rows0[row, pl.ds(j * 16, 16)] = (
                        rows0[row, pl.ds(j * 16, 16)] * nv)
            return carry2
        lax.fori_loop(0, RC // 16, scale2, 0)
        pltpu.sync_copy(rslice, acc.at[dstb2], add=True)

        plsc.subcore_barrier()
        for t in range(pl.cdiv(NU, 16)):
            u = sid + t * 16

            @pl.when(u < NU)
            def _():
                pltpu.sync_copy(acc.at[pl.ds(u * ZR, ZR)],
                                out_hbm.at[cid, pl.ds(u * ZR, ZR)])

    return functools.partial(
        pl.kernel, mesh=mesh, out_type=out_type, scratch_types=scratch,
    )(body)


def _table1_body(embp_ref, w_ref, out_ref):
    out_ref[0] = jnp.dot(embp_ref[...], w_ref[0], preferred_element_type=F32)


def _make_table1(R, D, H):
    return pl.pallas_call(
        _table1_body,
        grid=(R,),
        in_specs=[
            pl.BlockSpec((VP, D), lambda r: (0, 0)),
            pl.BlockSpec((1, D, H), lambda r: (r, 0, 0)),
        ],
        out_specs=pl.BlockSpec((1, VP, H), lambda r: (r, 0, 0)),
        out_shape=jax.ShapeDtypeStruct((R, VP, H), F32),
    )


def _make_prep(N, E, R):
    rows = E // 128

    def body(src_ref, dst_ref, rel_ref, tsrc_ref, g1_ref, g2_ref, key_ref):
        rel = rel_ref[...]
        g1_ref[...] = rel * VP + tsrc_ref[...]
        g2_ref[...] = rel * N + src_ref[...]
        key_ref[...] = dst_ref[...] * 128 + rel

    spec = pl.BlockSpec((rows, 128), lambda: (0, 0))
    return pl.pallas_call(
        body,
        in_specs=[spec] * 4,
        out_specs=[spec] * 3,
        out_shape=[jax.ShapeDtypeStruct((rows, 128), I32)] * 3,
    )


def _combine1_body(t_ref, p0_ref, p1_ref, embp_ref, root_ref, b_ref, out_ref):
    t = t_ref[...]
    oh = (t == lax.broadcasted_iota(jnp.int32, (1, VP), 1)).astype(F32)
    x = jnp.dot(oh, embp_ref[...], preferred_element_type=F32)
    acc = p0_ref[...] + p1_ref[...] + jnp.dot(x, root_ref[...], preferred_element_type=F32)
    out_ref[...] = jnp.maximum(acc + b_ref[...], 0.0)


def _make_combine1(N, D, H):
    return pl.pallas_call(
        _combine1_body,
        grid=(N // BN,),
        in_specs=[
            pl.BlockSpec((BN, 1), lambda i: (i, 0)),
            pl.BlockSpec((BN, H), lambda i: (i, 0)),
            pl.BlockSpec((BN, H), lambda i: (i, 0)),
            pl.BlockSpec((VP, D), lambda i: (0, 0)),
            pl.BlockSpec((D, H), lambda i: (0, 0)),
            pl.BlockSpec((1, H), lambda i: (0, 0)),
        ],
        out_specs=pl.BlockSpec((BN, H), lambda i: (i, 0)),
        out_shape=jax.ShapeDtypeStruct((N, H), F32),
    )


def _combine2_body(x_ref, p0_ref, p1_ref, root_ref, b_ref, out_ref):
    acc = p0_ref[...] + p1_ref[...] + jnp.dot(
        x_ref[...], root_ref[...], preferred_element_type=F32)
    out_ref[...] = jnp.maximum(acc + b_ref[...], 0.0)


def _make_combine2(N, H):
    return pl.pallas_call(
        _combine2_body,
        grid=(N // BN,),
        in_specs=[
            pl.BlockSpec((BN, H), lambda i: (i, 0)),
            pl.BlockSpec((BN, H), lambda i: (i, 0)),
            pl.BlockSpec((BN, H), lambda i: (i, 0)),
            pl.BlockSpec((H, H), lambda i: (0, 0)),
            pl.BlockSpec((1, H), lambda i: (0, 0)),
        ],
        out_specs=pl.BlockSpec((BN, H), lambda i: (i, 0)),
        out_shape=jax.ShapeDtypeStruct((N, H), F32),
    )


def _xr2_body(h_ref, w_ref, out_ref):
    out_ref[0] = jnp.dot(h_ref[...], w_ref[0], preferred_element_type=F32)


def _make_xr2(N, R, H):
    return pl.pallas_call(
        _xr2_body,
        grid=(R, N // BN),
        in_specs=[
            pl.BlockSpec((BN, H), lambda r, i: (i, 0)),
            pl.BlockSpec((1, H, H), lambda r, i: (r, 0, 0)),
        ],
        out_specs=pl.BlockSpec((1, BN, H), lambda r, i: (r, i, 0)),
        out_shape=jax.ShapeDtypeStruct((R, N, H), F32),
    )


def _pool_body(h_ref, bs_ref, aw_ref, lw_ref, lb_ref, out_ref):
    h = h_ref[...]
    logits = jnp.dot(h, aw_ref[...], preferred_element_type=F32)
    ohb = (bs_ref[...] == lax.broadcasted_iota(jnp.int32, (1, B), 1)).astype(F32)
    neg = jnp.float32(-1e30)
    m = jnp.max(jnp.where(ohb > 0, logits, neg), axis=0, keepdims=True)
    mg = jnp.sum(ohb * m, axis=1, keepdims=True)
    ex = jnp.exp(logits - mg)
    s = jnp.sum(ohb * ex, axis=0, keepdims=True)
    sg = jnp.sum(ohb * s, axis=1, keepdims=True)
    hw = h * (ex / sg)
    ge = lax.dot_general(ohb, hw, (((0,), (0,)), ((), ())),
                         preferred_element_type=F32)
    val = jax.nn.sigmoid(jnp.dot(ge, lw_ref[...], preferred_element_type=F32)
                         + lb_ref[...])
    out_ref[...] = jnp.broadcast_to(val, out_ref.shape)


def _make_pool(N, H):
    return pl.pallas_call(
        _pool_body,
        grid=(1,),
        in_specs=[
            pl.BlockSpec((N, H), lambda i: (0, 0)),
            pl.BlockSpec((N, 1), lambda i: (0, 0)),
            pl.BlockSpec((H, 1), lambda i: (0, 0)),
            pl.BlockSpec((H, 1), lambda i: (0, 0)),
            pl.BlockSpec((1, 1), lambda i: (0, 0)),
        ],
        out_specs=pl.BlockSpec((B, 128), lambda i: (0, 0)),
        out_shape=jax.ShapeDtypeStruct((B, 128), F32),
    )


def _norm_body(c0_ref, c1_ref, out_ref):
    out_ref[...] = 1.0 / jnp.maximum(c0_ref[...] + c1_ref[...], 1.0)


def _make_norm(rows, cols):
    return pl.pallas_call(
        _norm_body,
        grid=(1,),
        in_specs=[pl.BlockSpec((rows, cols), lambda i: (0, 0)),
                  pl.BlockSpec((rows, cols), lambda i: (0, 0))],
        out_specs=pl.BlockSpec((rows, cols), lambda i: (0, 0)),
        out_shape=jax.ShapeDtypeStruct((rows, cols), F32),
    )


def kernel(nodeTypes, edge_index, edge_attr, bs, emb, W1, root1, b1,
           W2, root2, b2, att_w, lin_w, lin_b):
    N, D = nodeTypes.shape[0], emb.shape[1]
    E = edge_attr.shape[0]
    R, H = W1.shape[0], W1.shape[2]
    V = emb.shape[0]

    src = edge_index[0]
    dst = edge_index[1]
    rel = edge_attr

    embpad = jnp.pad(emb, ((0, VP - V), (0, 0)))
    table1 = _make_table1(R, D, H)(embpad, W1)

    cntp, tsrc = _make_cnt(N, E, R, V)(src, dst, rel, nodeTypes)
    norm = _make_norm(N, 128)(cntp[0], cntp[1]).reshape(N * 128)

    e2 = (E // 128, 128)
    g1, g2, key = _make_prep(N, E, R)(src.reshape(e2), dst.reshape(e2),
                                      rel.reshape(e2), tsrc.reshape(e2))
    g1, g2, key = g1.reshape(E), g2.reshape(E), key.reshape(E)

    p1, norme = _make_edges(N, E, R, H, 1)(g1, dst, key, norm,
                                           table1.reshape(R * VP, H))

    types2d = nodeTypes.reshape(N, 1)
    h1 = _make_combine1(N, D, H)(types2d, p1[0], p1[1], embpad, root1,
                                 b1.reshape(1, H))

    xr2 = _make_xr2(N, R, H)(h1, W2)
    p2 = _make_edges(N, E, R, H, 2)(g2, dst, norme, xr2.reshape(R * N, H))

    h2 = _make_combine2(N, H)(h1, p2[0], p2[1], root2, b2.reshape(1, H))

    out = _make_pool(N, H)(h2, bs.reshape(N, 1), att_w.reshape(H, 1),
                           lin_w, lin_b.reshape(1, 1))
    return out[:, :1]


# speedup vs baseline: 1.1890x; 1.1890x over previous
import functools

import jax
import jax.numpy as jnp
from jax import lax
from jax.experimental import pallas as pl
from jax.experimental.pallas import tpu as pltpu
from jax.experimental.pallas import tpu_sc as plsc

F32 = jnp.float32
I32 = jnp.int32
VP = 128
B = 256
BN = 1000
NW = 32


def _make_cnt(N, E, R, V):
    KC = 128
    RC = 16
    EW = E // NW
    NCH = EW // KC
    assert EW == NCH * KC + RC
    ZR = 80
    NU = N // ZR
    mesh = plsc.VectorSubcoreMesh(core_axis_name="c", subcore_axis_name="s")

    @functools.partial(
        pl.kernel, mesh=mesh,
        out_type=(jax.ShapeDtypeStruct((2, N, 128), F32),
                  jax.ShapeDtypeStruct((E,), I32)),
        scratch_types=[
            pltpu.VMEM((KC,), I32),
            pltpu.VMEM((KC,), I32),
            pltpu.VMEM((KC,), I32),
            pltpu.VMEM((KC,), I32),
            pltpu.VMEM((RC,), I32),
            pltpu.VMEM((RC,), I32),
            pltpu.VMEM((RC,), I32),
            pltpu.VMEM((RC,), I32),
            pltpu.VMEM((KC, 128), F32),
            pltpu.VMEM((ZR, 128), F32),
            pltpu.VMEM_SHARED((N, 128), F32),
        ],
    )
    def k(src_hbm, dst_hbm, rel_hbm, types_hbm, out_hbm, tsrc_hbm,
          srcb, dstb, relb, tsrcb, srcb2, dstb2, relb2, tsrcb2, ohb, zb, acc):
        cid = lax.axis_index("c")
        sid = lax.axis_index("s")
        wid = sid * 2 + cid
        base = wid * EW
        zeros16 = jnp.zeros((16,), F32)
        iota = lax.iota(I32, 16)

        def zrow(i, carry):
            for j in range(8):
                zb[i, pl.ds(j * 16, 16)] = zeros16
            return carry
        lax.fori_loop(0, ZR, zrow, 0)

        def zoh(i, carry):
            for j in range(8):
                ohb[i, pl.ds(j * 16, 16)] = zeros16
            return carry
        lax.fori_loop(0, KC, zoh, 0)
        for t in range(pl.cdiv(NU, 16)):
            u = sid + t * 16

            @pl.when(u < NU)
            def _():
                pltpu.sync_copy(zb, acc.at[pl.ds(u * ZR, ZR)])
        plsc.subcore_barrier()

        def do_chunk(off, sz, sb, db, rb, tb):
            pltpu.sync_copy(src_hbm.at[pl.ds(off, sz)], sb)
            pltpu.sync_copy(dst_hbm.at[pl.ds(off, sz)], db)
            pltpu.sync_copy(rel_hbm.at[pl.ds(off, sz)], rb)
            pltpu.sync_copy(types_hbm.at[sb], tb)
            pltpu.sync_copy(tb, tsrc_hbm.at[pl.ds(off, sz)])

            def grp(j, carry2):
                r16 = rb[pl.ds(j * 16, 16)]
                for kk in range(16):
                    rv = jnp.full((16,), r16[kk], I32)
                    ohb[j * 16 + kk, pl.ds(0, 16)] = (
                        jnp.where(iota == rv, 1.0, 0.0).astype(F32))
                return carry2
            lax.fori_loop(0, sz // 16, grp, 0)
            oslice = ohb.at[pl.ds(0, sz)] if sz != KC else ohb
            pltpu.sync_copy(oslice, acc.at[db], add=True)

        def chunk(c, carry):
            do_chunk(base + c * KC, KC, srcb, dstb, relb, tsrcb)
            return carry
        lax.fori_loop(0, NCH, chunk, 0)
        do_chunk(base + NCH * KC, RC, srcb2, dstb2, relb2, tsrcb2)

        plsc.subcore_barrier()
        for t in range(pl.cdiv(NU, 16)):
            u = sid + t * 16

            @pl.when(u < NU)
            def _():
                pltpu.sync_copy(acc.at[pl.ds(u * ZR, ZR)],
                                out_hbm.at[cid, pl.ds(u * ZR, ZR)])

    return k


def _make_edges(N, E, R, H, layer):
    KC = 128
    RC = 16
    EW = E // NW
    NCH = EW // KC
    assert EW == NCH * KC + RC
    ZR = 80
    NU = N // ZR
    mesh = plsc.VectorSubcoreMesh(core_axis_name="c", subcore_axis_name="s")

    scratch = [
        pltpu.VMEM((KC,), I32),
        pltpu.VMEM((KC,), I32),
        pltpu.VMEM((KC,), I32),
        pltpu.VMEM((KC,), F32),
        pltpu.VMEM((KC,), F32),
        pltpu.VMEM((RC,), I32),
        pltpu.VMEM((RC,), I32),
        pltpu.VMEM((RC,), F32),
        pltpu.VMEM((KC, H), F32),
        pltpu.VMEM((KC, H), F32),
        pltpu.VMEM((ZR, H), F32),
        pltpu.VMEM_SHARED((N, H), F32),
        pltpu.SemaphoreType.DMA,
        pltpu.SemaphoreType.DMA,
    ]
    if layer == 1:
        scratch += [pltpu.VMEM((KC,), I32), pltpu.VMEM((RC,), I32)]
        out_type = (jax.ShapeDtypeStruct((2, N, H), F32),
                    jax.ShapeDtypeStruct((E,), F32))
    else:
        out_type = jax.ShapeDtypeStruct((2, N, H), F32)

    def body(*args):
        if layer == 1:
            (gidx_hbm, dst_hbm, key_hbm, nflat_hbm, table_hbm,
             out_hbm, norme_hbm, g0, g1, dstb, n0, n1, gidxb2, dstb2, nrmb2,
             rows0, rows1, zb, acc, sem0, sem1, keyb, keyb2) = args
        else:
            (gidx_hbm, dst_hbm, norme_hbm, table_hbm,
             out_hbm, g0, g1, dstb, n0, n1, gidxb2, dstb2, nrmb2,
             rows0, rows1, zb, acc, sem0, sem1) = args
        gb_ = (g0, g1)
        nb_ = (n0, n1)
        rows_ = (rows0, rows1)
        sem_ = (sem0, sem1)
        cid = lax.axis_index("c")
        sid = lax.axis_index("s")
        wid = sid * 2 + cid
        base = wid * EW
        zeros16 = jnp.zeros((16,), F32)

        def zrow(i, carry):
            for j in range(H // 16):
                zb[i, pl.ds(j * 16, 16)] = zeros16
            return carry
        lax.fori_loop(0, ZR, zrow, 0)
        for t in range(pl.cdiv(NU, 16)):
            u = sid + t * 16

            @pl.when(u < NU)
            def _():
                pltpu.sync_copy(zb, acc.at[pl.ds(u * ZR, ZR)])
        plsc.subcore_barrier()

        def load_idx(off, bn):
            pltpu.sync_copy(gidx_hbm.at[pl.ds(off, KC)], gb_[bn])
            if layer == 1:
                pltpu.sync_copy(key_hbm.at[pl.ds(off, KC)], keyb)
                pltpu.sync_copy(nflat_hbm.at[keyb], nb_[bn])
                pltpu.sync_copy(nb_[bn], norme_hbm.at[pl.ds(off, KC)])
            else:
                pltpu.sync_copy(norme_hbm.at[pl.ds(off, KC)], nb_[bn])
            pltpu.async_copy(table_hbm.at[gb_[bn]], rows_[bn], sem_[bn])

        def finish_chunk(off, bi):
            pltpu.make_async_copy(table_hbm.at[gb_[bi]], rows_[bi],
                                  sem_[bi]).wait()
            rows = rows_[bi]
            nb = nb_[bi]

            def scale(jg, carry2):
                nv16 = nb[pl.ds(jg * 16, 16)]
                for kk in range(16):
                    row = jg * 16 + kk
                    nv = jnp.full((16,), nv16[kk], F32)
                    for j in range(H // 16):
                        rows[row, pl.ds(j * 16, 16)] = (
                            rows[row, pl.ds(j * 16, 16)] * nv)
                return carry2
            lax.fori_loop(0, KC // 16, scale, 0)
            pltpu.sync_copy(dst_hbm.at[pl.ds(off, KC)], dstb)
            pltpu.sync_copy(rows, acc.at[dstb], add=True)

        load_idx(base, 0)

        def pair(o, carry):
            for b in range(2):
                i = 2 * o + b
                load_idx(base + (i + 1) * KC, 1 - b)
                finish_chunk(base + i * KC, b)
            return carry
        lax.fori_loop(0, (NCH - 1) // 2, pair, 0)
        if (NCH - 1) % 2:
            i = NCH - 2
            load_idx(base + (i + 1) * KC, (i + 1) % 2)
            finish_chunk(base + i * KC, i % 2)
        finish_chunk(base + (NCH - 1) * KC, (NCH - 1) % 2)

        roff = base + NCH * KC
        pltpu.sync_copy(gidx_hbm.at[pl.ds(roff, RC)], gidxb2)
        pltpu.sync_copy(dst_hbm.at[pl.ds(roff, RC)], dstb2)
        if layer == 1:
            pltpu.sync_copy(key_hbm.at[pl.ds(roff, RC)], keyb2)
            pltpu.sync_copy(nflat_hbm.at[keyb2], nrmb2)
            pltpu.sync_copy(nrmb2, norme_hbm.at[pl.ds(roff, RC)])
        else:
            pltpu.sync_copy(norme_hbm.at[pl.ds(roff, RC)], nrmb2)
        rslice = rows0.at[pl.ds(0, RC)]
        pltpu.sync_copy(table_hbm.at[gidxb2], rslice)

        def scale2(jg, carry2):
            nv16 = nrmb2[pl.ds(jg * 16, 16)]
            for kk in range(16):
                row = jg * 16 + kk
                nv = jnp.full((16,), nv16[kk], F32)
                for j in range(H // 16):
                    rows0[row, pl.ds(j * 16, 16)] = (
                        rows0[row, pl.ds(j * 16, 16)] * nv)
            return carry2
        lax.fori_loop(0, RC // 16, scale2, 0)
        pltpu.sync_copy(rslice, acc.at[dstb2], add=True)

        plsc.subcore_barrier()
        for t in range(pl.cdiv(NU, 16)):
            u = sid + t * 16

            @pl.when(u < NU)
            def _():
                pltpu.sync_copy(acc.at[pl.ds(u * ZR, ZR)],
                                out_hbm.at[cid, pl.ds(u * ZR, ZR)])

    return functools.partial(
        pl.kernel, mesh=mesh, out_type=out_type, scratch_types=scratch,
    )(body)


def _table1_body(embp_ref, w_ref, out_ref):
    out_ref[0] = jnp.dot(embp_ref[...], w_ref[0], preferred_element_type=F32)


def _make_table1(R, D, H):
    return pl.pallas_call(
        _table1_body,
        grid=(R,),
        in_specs=[
            pl.BlockSpec((VP, D), lambda r: (0, 0)),
            pl.BlockSpec((1, D, H), lambda r: (r, 0, 0)),
        ],
        out_specs=pl.BlockSpec((1, VP, H), lambda r: (r, 0, 0)),
        out_shape=jax.ShapeDtypeStruct((R, VP, H), F32),
    )


def _make_prep(N, E, R):
    rows = E // 128

    def body(src_ref, dst_ref, rel_ref, tsrc_ref, g1_ref, g2_ref, key_ref):
        rel = rel_ref[...]
        g1_ref[...] = rel * VP + tsrc_ref[...]
        g2_ref[...] = rel * N + src_ref[...]
        key_ref[...] = dst_ref[...] * 128 + rel

    spec = pl.BlockSpec((rows, 128), lambda: (0, 0))
    return pl.pallas_call(
        body,
        in_specs=[spec] * 4,
        out_specs=[spec] * 3,
        out_shape=[jax.ShapeDtypeStruct((rows, 128), I32)] * 3,
    )


def _combine1_body(t_ref, p0_ref, p1_ref, embp_ref, root_ref, b_ref, out_ref):
    t = t_ref[...]
    oh = (t == lax.broadcasted_iota(jnp.int32, (1, VP), 1)).astype(F32)
    x = jnp.dot(oh, embp_ref[...], preferred_element_type=F32)
    acc = p0_ref[...] + p1_ref[...] + jnp.dot(x, root_ref[...], preferred_element_type=F32)
    out_ref[...] = jnp.maximum(acc + b_ref[...], 0.0)


def _make_combine1(N, D, H):
    return pl.pallas_call(
        _combine1_body,
        grid=(N // BN,),
        in_specs=[
            pl.BlockSpec((BN, 1), lambda i: (i, 0)),
            pl.BlockSpec((BN, H), lambda i: (i, 0)),
            pl.BlockSpec((BN, H), lambda i: (i, 0)),
            pl.BlockSpec((VP, D), lambda i: (0, 0)),
            pl.BlockSpec((D, H), lambda i: (0, 0)),
            pl.BlockSpec((1, H), lambda i: (0, 0)),
        ],
        out_specs=pl.BlockSpec((BN, H), lambda i: (i, 0)),
        out_shape=jax.ShapeDtypeStruct((N, H), F32),
    )


def _combine2_body(x_ref, p0_ref, p1_ref, root_ref, b_ref, out_ref):
    acc = p0_ref[...] + p1_ref[...] + jnp.dot(
        x_ref[...], root_ref[...], preferred_element_type=F32)
    out_ref[...] = jnp.maximum(acc + b_ref[...], 0.0)


def _make_combine2(N, H):
    return pl.pallas_call(
        _combine2_body,
        grid=(N // BN,),
        in_specs=[
            pl.BlockSpec((BN, H), lambda i: (i, 0)),
            pl.BlockSpec((BN, H), lambda i: (i, 0)),
            pl.BlockSpec((BN, H), lambda i: (i, 0)),
            pl.BlockSpec((H, H), lambda i: (0, 0)),
            pl.BlockSpec((1, H), lambda i: (0, 0)),
        ],
        out_specs=pl.BlockSpec((BN, H), lambda i: (i, 0)),
        out_shape=jax.ShapeDtypeStruct((N, H), F32),
    )


def _xr2_body(h_ref, w_ref, out_ref):
    out_ref[0] = jnp.dot(h_ref[...], w_ref[0], preferred_element_type=F32)


def _make_xr2(N, R, H):
    return pl.pallas_call(
        _xr2_body,
        grid=(R, N // BN),
        in_specs=[
            pl.BlockSpec((BN, H), lambda r, i: (i, 0)),
            pl.BlockSpec((1, H, H), lambda r, i: (r, 0, 0)),
        ],
        out_specs=pl.BlockSpec((1, BN, H), lambda r, i: (r, i, 0)),
        out_shape=jax.ShapeDtypeStruct((R, N, H), F32),
    )


def _pool_body(h_ref, bs_ref, aw_ref, lw_ref, lb_ref, out_ref):
    h = h_ref[...]
    logits = jnp.dot(h, aw_ref[...], preferred_element_type=F32)
    ohb = (bs_ref[...] == lax.broadcasted_iota(jnp.int32, (1, B), 1)).astype(F32)
    neg = jnp.float32(-1e30)
    m = jnp.max(jnp.where(ohb > 0, logits, neg), axis=0, keepdims=True)
    mg = jnp.sum(ohb * m, axis=1, keepdims=True)
    ex = jnp.exp(logits - mg)
    s = jnp.sum(ohb * ex, axis=0, keepdims=True)
    sg = jnp.sum(ohb * s, axis=1, keepdims=True)
    hw = h * (ex / sg)
    ge = lax.dot_general(ohb, hw, (((0,), (0,)), ((), ())),
                         preferred_element_type=F32)
    val = jax.nn.sigmoid(jnp.dot(ge, lw_ref[...], preferred_element_type=F32)
                         + lb_ref[...])
    out_ref[...] = jnp.broadcast_to(val, out_ref.shape)


def _make_pool(N, H):
    return pl.pallas_call(
        _pool_body,
        grid=(1,),
        in_specs=[
            pl.BlockSpec((N, H), lambda i: (0, 0)),
            pl.BlockSpec((N, 1), lambda i: (0, 0)),
            pl.BlockSpec((H, 1), lambda i: (0, 0)),
            pl.BlockSpec((H, 1), lambda i: (0, 0)),
            pl.BlockSpec((1, 1), lambda i: (0, 0)),
        ],
        out_specs=pl.BlockSpec((B, 128), lambda i: (0, 0)),
        out_shape=jax.ShapeDtypeStruct((B, 128), F32),
    )


def _norm_body(c0_ref, c1_ref, out_ref):
    out_ref[...] = 1.0 / jnp.maximum(c0_ref[...] + c1_ref[...], 1.0)


def _make_norm(rows, cols):
    return pl.pallas_call(
        _norm_body,
        grid=(1,),
        in_specs=[pl.BlockSpec((rows, cols), lambda i: (0, 0)),
                  pl.BlockSpec((rows, cols), lambda i: (0, 0))],
        out_specs=pl.BlockSpec((rows, cols), lambda i: (0, 0)),
        out_shape=jax.ShapeDtypeStruct((rows, cols), F32),
    )


def kernel(nodeTypes, edge_index, edge_attr, bs, emb, W1, root1, b1,
           W2, root2, b2, att_w, lin_w, lin_b):
    N, D = nodeTypes.shape[0], emb.shape[1]
    E = edge_attr.shape[0]
    R, H = W1.shape[0], W1.shape[2]
    V = emb.shape[0]

    src = edge_index[0]
    dst = edge_index[1]
    rel = edge_attr

    embpad = jnp.pad(emb, ((0, VP - V), (0, 0)))
    table1 = _make_table1(R, D, H)(embpad, W1)

    cntp, tsrc = _make_cnt(N, E, R, V)(src, dst, rel, nodeTypes)
    norm = _make_norm(N, 128)(cntp[0], cntp[1]).reshape(N * 128)

    e2 = (E // 128, 128)
    g1, g2, key = _make_prep(N, E, R)(src.reshape(e2), dst.reshape(e2),
                                      rel.reshape(e2), tsrc.reshape(e2))
    g1, g2, key = g1.reshape(E), g2.reshape(E), key.reshape(E)

    p1, norme = _make_edges(N, E, R, H, 1)(g1, dst, key, norm,
                                           table1.reshape(R * VP, H))

    types2d = nodeTypes.reshape(N, 1)
    h1 = _make_combine1(N, D, H)(types2d, p1[0], p1[1], embpad, root1,
                                 b1.reshape(1, H))

    xr2 = _make_xr2(N, R, H)(h1, W2)
    p2 = _make_edges(N, E, R, H, 2)(g2, dst, norme, xr2.reshape(R * N, H))

    h2 = _make_combine2(N, H)(h1, p2[0], p2[1], root2, b2.reshape(1, H))

    out = _make_pool(N, H)(h2, bs.reshape(N, 1), att_w.reshape(H, 1),
                           lin_w, lin_b.reshape(1, 1))
    return out[:, :1]


# speedup vs baseline: 1.3286x; 1.1174x over previous
import functools

import jax
import jax.numpy as jnp
from jax import lax
from jax.experimental import pallas as pl
from jax.experimental.pallas import tpu as pltpu
from jax.experimental.pallas import tpu_sc as plsc

F32 = jnp.float32
I32 = jnp.int32
VP = 128
B = 256
BN = 1000
NW = 32


def _make_cnt(N, E, R, V):
    KC = 128
    RC = 16
    EW = E // NW
    NCH = EW // KC
    assert EW == NCH * KC + RC
    ZR = 80
    NU = N // ZR
    mesh = plsc.VectorSubcoreMesh(core_axis_name="c", subcore_axis_name="s")

    @functools.partial(
        pl.kernel, mesh=mesh,
        out_type=(jax.ShapeDtypeStruct((2, N, 128), F32),
                  jax.ShapeDtypeStruct((E,), I32)),
        scratch_types=[
            pltpu.VMEM((KC,), I32),
            pltpu.VMEM((KC,), I32),
            pltpu.VMEM((KC,), I32),
            pltpu.VMEM((KC,), I32),
            pltpu.VMEM((KC,), I32),
            pltpu.VMEM((KC,), I32),
            pltpu.VMEM((KC,), I32),
            pltpu.VMEM((RC,), I32),
            pltpu.VMEM((RC,), I32),
            pltpu.VMEM((RC,), I32),
            pltpu.VMEM((RC,), I32),
            pltpu.VMEM((KC, 128), F32),
            pltpu.VMEM((ZR, 128), F32),
            pltpu.VMEM_SHARED((N, 128), F32),
            pltpu.SemaphoreType.DMA,
            pltpu.SemaphoreType.DMA,
        ],
    )
    def k(src_hbm, dst_hbm, rel_hbm, types_hbm, out_hbm, tsrc_hbm,
          sb0, sb1, db0, db1, rb0, rb1, tsrcb, srcb2, dstb2, relb2, tsrcb2,
          ohb, zb, acc, lsem0, lsem1):
        sb_ = (sb0, sb1)
        db_ = (db0, db1)
        rb_ = (rb0, rb1)
        lsem_ = (lsem0, lsem1)
        cid = lax.axis_index("c")
        sid = lax.axis_index("s")
        wid = sid * 2 + cid
        base = wid * EW
        zeros16 = jnp.zeros((16,), F32)
        iota = lax.iota(I32, 16)

        def zrow(i, carry):
            for j in range(8):
                zb[i, pl.ds(j * 16, 16)] = zeros16
            return carry
        lax.fori_loop(0, ZR, zrow, 0)

        def zoh(i, carry):
            for j in range(8):
                ohb[i, pl.ds(j * 16, 16)] = zeros16
            return carry
        lax.fori_loop(0, KC, zoh, 0)
        for t in range(pl.cdiv(NU, 16)):
            u = sid + t * 16

            @pl.when(u < NU)
            def _():
                pltpu.sync_copy(zb, acc.at[pl.ds(u * ZR, ZR)])
        plsc.subcore_barrier()

        def start3(off, bn):
            pltpu.async_copy(src_hbm.at[pl.ds(off, KC)], sb_[bn], lsem_[bn])
            pltpu.async_copy(dst_hbm.at[pl.ds(off, KC)], db_[bn], lsem_[bn])
            pltpu.async_copy(rel_hbm.at[pl.ds(off, KC)], rb_[bn], lsem_[bn])

        def wait3(off, bi):
            pltpu.make_async_copy(src_hbm.at[pl.ds(off, KC)], sb_[bi],
                                  lsem_[bi]).wait()
            pltpu.make_async_copy(dst_hbm.at[pl.ds(off, KC)], db_[bi],
                                  lsem_[bi]).wait()
            pltpu.make_async_copy(rel_hbm.at[pl.ds(off, KC)], rb_[bi],
                                  lsem_[bi]).wait()

        def work(off, bi):
            wait3(off, bi)
            pltpu.sync_copy(types_hbm.at[sb_[bi]], tsrcb)
            pltpu.sync_copy(tsrcb, tsrc_hbm.at[pl.ds(off, KC)])
            rb = rb_[bi]

            def grp(j, carry2):
                r16 = rb[pl.ds(j * 16, 16)]
                for kk in range(16):
                    rv = jnp.full((16,), r16[kk], I32)
                    ohb[j * 16 + kk, pl.ds(0, 16)] = (
                        jnp.where(iota == rv, 1.0, 0.0).astype(F32))
                return carry2
            lax.fori_loop(0, KC // 16, grp, 0)
            pltpu.sync_copy(ohb, acc.at[db_[bi]], add=True)

        start3(base, 0)

        def pair(o, carry):
            for b in range(2):
                i = 2 * o + b
                off = base + i * KC

                @pl.when(i + 1 < NCH)
                def _():
                    start3(base + (i + 1) * KC, 1 - b)
                work(off, b)
            return carry
        lax.fori_loop(0, NCH // 2, pair, 0)

        roff = base + NCH * KC
        pltpu.sync_copy(src_hbm.at[pl.ds(roff, RC)], srcb2)
        pltpu.sync_copy(dst_hbm.at[pl.ds(roff, RC)], dstb2)
        pltpu.sync_copy(rel_hbm.at[pl.ds(roff, RC)], relb2)
        pltpu.sync_copy(types_hbm.at[srcb2], tsrcb2)
        pltpu.sync_copy(tsrcb2, tsrc_hbm.at[pl.ds(roff, RC)])

        def grp2(j, carry2):
            r16 = relb2[pl.ds(j * 16, 16)]
            for kk in range(16):
                rv = jnp.full((16,), r16[kk], I32)
                ohb[j * 16 + kk, pl.ds(0, 16)] = (
                    jnp.where(iota == rv, 1.0, 0.0).astype(F32))
            return carry2
        lax.fori_loop(0, RC // 16, grp2, 0)
        pltpu.sync_copy(ohb.at[pl.ds(0, RC)], acc.at[dstb2], add=True)

        plsc.subcore_barrier()
        for t in range(pl.cdiv(NU, 16)):
            u = sid + t * 16

            @pl.when(u < NU)
            def _():
                pltpu.sync_copy(acc.at[pl.ds(u * ZR, ZR)],
                                out_hbm.at[cid, pl.ds(u * ZR, ZR)])

    return k


def _make_edges(N, E, R, H, layer):
    KC = 128
    RC = 16
    EW = E // NW
    NCH = EW // KC
    assert EW == NCH * KC + RC
    ZR = 80
    NU = N // ZR
    mesh = plsc.VectorSubcoreMesh(core_axis_name="c", subcore_axis_name="s")

    scratch = [
        pltpu.VMEM((KC,), I32),
        pltpu.VMEM((KC,), I32),
        pltpu.VMEM((KC,), I32),
        pltpu.VMEM((KC,), F32),
        pltpu.VMEM((KC,), F32),
        pltpu.VMEM((RC,), I32),
        pltpu.VMEM((RC,), I32),
        pltpu.VMEM((RC,), F32),
        pltpu.VMEM((KC, H), F32),
        pltpu.VMEM((KC, H), F32),
        pltpu.VMEM((ZR, H), F32),
        pltpu.VMEM_SHARED((N, H), F32),
        pltpu.SemaphoreType.DMA,
        pltpu.SemaphoreType.DMA,
    ]
    if layer == 1:
        scratch += [pltpu.VMEM((KC,), I32), pltpu.VMEM((RC,), I32)]
        out_type = (jax.ShapeDtypeStruct((2, N, H), F32),
                    jax.ShapeDtypeStruct((E,), F32))
    else:
        out_type = jax.ShapeDtypeStruct((2, N, H), F32)

    def body(*args):
        if layer == 1:
            (gidx_hbm, dst_hbm, key_hbm, nflat_hbm, table_hbm,
             out_hbm, norme_hbm, g0, g1, dstb, n0, n1, gidxb2, dstb2, nrmb2,
             rows0, rows1, zb, acc, sem0, sem1, keyb, keyb2) = args
        else:
            (gidx_hbm, dst_hbm, norme_hbm, table_hbm,
             out_hbm, g0, g1, dstb, n0, n1, gidxb2, dstb2, nrmb2,
             rows0, rows1, zb, acc, sem0, sem1) = args
        gb_ = (g0, g1)
        nb_ = (n0, n1)
        rows_ = (rows0, rows1)
        sem_ = (sem0, sem1)
        cid = lax.axis_index("c")
        sid = lax.axis_index("s")
        wid = sid * 2 + cid
        base = wid * EW
        zeros16 = jnp.zeros((16,), F32)

        def zrow(i, carry):
            for j in range(H // 16):
                zb[i, pl.ds(j * 16, 16)] = zeros16
            return carry
        lax.fori_loop(0, ZR, zrow, 0)
        for t in range(pl.cdiv(NU, 16)):
            u = sid + t * 16

            @pl.when(u < NU)
            def _():
                pltpu.sync_copy(zb, acc.at[pl.ds(u * ZR, ZR)])
        plsc.subcore_barrier()

        def load_idx(off, bn):
            pltpu.sync_copy(gidx_hbm.at[pl.ds(off, KC)], gb_[bn])
            if layer == 1:
                pltpu.sync_copy(key_hbm.at[pl.ds(off, KC)], keyb)
                pltpu.sync_copy(nflat_hbm.at[keyb], nb_[bn])
                pltpu.sync_copy(nb_[bn], norme_hbm.at[pl.ds(off, KC)])
            else:
                pltpu.sync_copy(norme_hbm.at[pl.ds(off, KC)], nb_[bn])
            pltpu.async_copy(table_hbm.at[gb_[bn]], rows_[bn], sem_[bn])

        def finish_chunk(off, bi):
            pltpu.make_async_copy(table_hbm.at[gb_[bi]], rows_[bi],
                                  sem_[bi]).wait()
            rows = rows_[bi]
            nb = nb_[bi]

            def scale(jg, carry2):
                nv16 = nb[pl.ds(jg * 16, 16)]
                for kk in range(16):
                    row = jg * 16 + kk
                    nv = jnp.full((16,), nv16[kk], F32)
                    for j in range(H // 16):
                        rows[row, pl.ds(j * 16, 16)] = (
                            rows[row, pl.ds(j * 16, 16)] * nv)
                return carry2
            lax.fori_loop(0, KC // 16, scale, 0)
            pltpu.sync_copy(dst_hbm.at[pl.ds(off, KC)], dstb)
            pltpu.sync_copy(rows, acc.at[dstb], add=True)

        load_idx(base, 0)

        def pair(o, carry):
            for b in range(2):
                i = 2 * o + b
                load_idx(base + (i + 1) * KC, 1 - b)
                finish_chunk(base + i * KC, b)
            return carry
        lax.fori_loop(0, (NCH - 1) // 2, pair, 0)
        if (NCH - 1) % 2:
            i = NCH - 2
            load_idx(base + (i + 1) * KC, (i + 1) % 2)
            finish_chunk(base + i * KC, i % 2)
        finish_chunk(base + (NCH - 1) * KC, (NCH - 1) % 2)

        roff = base + NCH * KC
        pltpu.sync_copy(gidx_hbm.at[pl.ds(roff, RC)], gidxb2)
        pltpu.sync_copy(dst_hbm.at[pl.ds(roff, RC)], dstb2)
        if layer == 1:
            pltpu.sync_copy(key_hbm.at[pl.ds(roff, RC)], keyb2)
            pltpu.sync_copy(nflat_hbm.at[keyb2], nrmb2)
            pltpu.sync_copy(nrmb2, norme_hbm.at[pl.ds(roff, RC)])
        else:
            pltpu.sync_copy(norme_hbm.at[pl.ds(roff, RC)], nrmb2)
        rslice = rows0.at[pl.ds(0, RC)]
        pltpu.sync_copy(table_hbm.at[gidxb2], rslice)

        def scale2(jg, carry2):
            nv16 = nrmb2[pl.ds(jg * 16, 16)]
            for kk in range(16):
                row = jg * 16 + kk
                nv = jnp.full((16,), nv16[kk], F32)
                for j in range(H // 16):
                    rows0[row, pl.ds(j * 16, 16)] = (
                        rows0[row, pl.ds(j * 16, 16)] * nv)
            return carry2
        lax.fori_loop(0, RC // 16, scale2, 0)
        pltpu.sync_copy(rslice, acc.at[dstb2], add=True)

        plsc.subcore_barrier()
        for t in range(pl.cdiv(NU, 16)):
            u = sid + t * 16

            @pl.when(u < NU)
            def _():
                pltpu.sync_copy(acc.at[pl.ds(u * ZR, ZR)],
                                out_hbm.at[cid, pl.ds(u * ZR, ZR)])

    return functools.partial(
        pl.kernel, mesh=mesh, out_type=out_type, scratch_types=scratch,
    )(body)


def _table1_body(embp_ref, w_ref, out_ref):
    out_ref[0] = jnp.dot(embp_ref[...], w_ref[0], preferred_element_type=F32)


def _make_table1(R, D, H):
    return pl.pallas_call(
        _table1_body,
        grid=(R,),
        in_specs=[
            pl.BlockSpec((VP, D), lambda r: (0, 0)),
            pl.BlockSpec((1, D, H), lambda r: (r, 0, 0)),
        ],
        out_specs=pl.BlockSpec((1, VP, H), lambda r: (r, 0, 0)),
        out_shape=jax.ShapeDtypeStruct((R, VP, H), F32),
    )


def _make_prep(N, E, R):
    rows = E // 128

    def body(src_ref, dst_ref, rel_ref, tsrc_ref, g1_ref, g2_ref, key_ref):
        rel = rel_ref[...]
        g1_ref[...] = rel * VP + tsrc_ref[...]
        g2_ref[...] = rel * N + src_ref[...]
        key_ref[...] = dst_ref[...] * 128 + rel

    spec = pl.BlockSpec((rows, 128), lambda: (0, 0))
    return pl.pallas_call(
        body,
        in_specs=[spec] * 4,
        out_specs=[spec] * 3,
        out_shape=[jax.ShapeDtypeStruct((rows, 128), I32)] * 3,
    )


def _combine1_body(t_ref, p0_ref, p1_ref, embp_ref, root_ref, b_ref, out_ref):
    t = t_ref[...]
    oh = (t == lax.broadcasted_iota(jnp.int32, (1, VP), 1)).astype(F32)
    x = jnp.dot(oh, embp_ref[...], preferred_element_type=F32)
    acc = p0_ref[...] + p1_ref[...] + jnp.dot(x, root_ref[...], preferred_element_type=F32)
    out_ref[...] = jnp.maximum(acc + b_ref[...], 0.0)


def _make_combine1(N, D, H):
    return pl.pallas_call(
        _combine1_body,
        grid=(N // BN,),
        in_specs=[
            pl.BlockSpec((BN, 1), lambda i: (i, 0)),
            pl.BlockSpec((BN, H), lambda i: (i, 0)),
            pl.BlockSpec((BN, H), lambda i: (i, 0)),
            pl.BlockSpec((VP, D), lambda i: (0, 0)),
            pl.BlockSpec((D, H), lambda i: (0, 0)),
            pl.BlockSpec((1, H), lambda i: (0, 0)),
        ],
        out_specs=pl.BlockSpec((BN, H), lambda i: (i, 0)),
        out_shape=jax.ShapeDtypeStruct((N, H), F32),
    )


def _combine2_body(x_ref, p0_ref, p1_ref, root_ref, b_ref, out_ref):
    acc = p0_ref[...] + p1_ref[...] + jnp.dot(
        x_ref[...], root_ref[...], preferred_element_type=F32)
    out_ref[...] = jnp.maximum(acc + b_ref[...], 0.0)


def _make_combine2(N, H):
    return pl.pallas_call(
        _combine2_body,
        grid=(N // BN,),
        in_specs=[
            pl.BlockSpec((BN, H), lambda i: (i, 0)),
            pl.BlockSpec((BN, H), lambda i: (i, 0)),
            pl.BlockSpec((BN, H), lambda i: (i, 0)),
            pl.BlockSpec((H, H), lambda i: (0, 0)),
            pl.BlockSpec((1, H), lambda i: (0, 0)),
        ],
        out_specs=pl.BlockSpec((BN, H), lambda i: (i, 0)),
        out_shape=jax.ShapeDtypeStruct((N, H), F32),
    )


def _xr2_body(h_ref, w_ref, out_ref):
    out_ref[0] = jnp.dot(h_ref[...], w_ref[0], preferred_element_type=F32)


def _make_xr2(N, R, H):
    return pl.pallas_call(
        _xr2_body,
        grid=(R, N // BN),
        in_specs=[
            pl.BlockSpec((BN, H), lambda r, i: (i, 0)),
            pl.BlockSpec((1, H, H), lambda r, i: (r, 0, 0)),
        ],
        out_specs=pl.BlockSpec((1, BN, H), lambda r, i: (r, i, 0)),
        out_shape=jax.ShapeDtypeStruct((R, N, H), F32),
    )


def _pool_body(h_ref, bs_ref, aw_ref, lw_ref, lb_ref, out_ref):
    h = h_ref[...]
    logits = jnp.dot(h, aw_ref[...], preferred_element_type=F32)
    ohb = (bs_ref[...] == lax.broadcasted_iota(jnp.int32, (1, B), 1)).astype(F32)
    neg = jnp.float32(-1e30)
    m = jnp.max(jnp.where(ohb > 0, logits, neg), axis=0, keepdims=True)
    mg = jnp.sum(ohb * m, axis=1, keepdims=True)
    ex = jnp.exp(logits - mg)
    s = jnp.sum(ohb * ex, axis=0, keepdims=True)
    sg = jnp.sum(ohb * s, axis=1, keepdims=True)
    hw = h * (ex / sg)
    ge = lax.dot_general(ohb, hw, (((0,), (0,)), ((), ())),
                         preferred_element_type=F32)
    val = jax.nn.sigmoid(jnp.dot(ge, lw_ref[...], preferred_element_type=F32)
                         + lb_ref[...])
    out_ref[...] = jnp.broadcast_to(val, out_ref.shape)


def _make_pool(N, H):
    return pl.pallas_call(
        _pool_body,
        grid=(1,),
        in_specs=[
            pl.BlockSpec((N, H), lambda i: (0, 0)),
            pl.BlockSpec((N, 1), lambda i: (0, 0)),
            pl.BlockSpec((H, 1), lambda i: (0, 0)),
            pl.BlockSpec((H, 1), lambda i: (0, 0)),
            pl.BlockSpec((1, 1), lambda i: (0, 0)),
        ],
        out_specs=pl.BlockSpec((B, 128), lambda i: (0, 0)),
        out_shape=jax.ShapeDtypeStruct((B, 128), F32),
    )


def _norm_body(c0_ref, c1_ref, out_ref):
    out_ref[...] = 1.0 / jnp.maximum(c0_ref[...] + c1_ref[...], 1.0)


def _make_norm(rows, cols):
    return pl.pallas_call(
        _norm_body,
        grid=(1,),
        in_specs=[pl.BlockSpec((rows, cols), lambda i: (0, 0)),
                  pl.BlockSpec((rows, cols), lambda i: (0, 0))],
        out_specs=pl.BlockSpec((rows, cols), lambda i: (0, 0)),
        out_shape=jax.ShapeDtypeStruct((rows, cols), F32),
    )


def kernel(nodeTypes, edge_index, edge_attr, bs, emb, W1, root1, b1,
           W2, root2, b2, att_w, lin_w, lin_b):
    N, D = nodeTypes.shape[0], emb.shape[1]
    E = edge_attr.shape[0]
    R, H = W1.shape[0], W1.shape[2]
    V = emb.shape[0]

    src = edge_index[0]
    dst = edge_index[1]
    rel = edge_attr

    embpad = jnp.pad(emb, ((0, VP - V), (0, 0)))
    table1 = _make_table1(R, D, H)(embpad, W1)

    cntp, tsrc = _make_cnt(N, E, R, V)(src, dst, rel, nodeTypes)
    norm = _make_norm(N, 128)(cntp[0], cntp[1]).reshape(N * 128)

    e2 = (E // 128, 128)
    g1, g2, key = _make_prep(N, E, R)(src.reshape(e2), dst.reshape(e2),
                                      rel.reshape(e2), tsrc.reshape(e2))
    g1, g2, key = g1.reshape(E), g2.reshape(E), key.reshape(E)

    p1, norme = _make_edges(N, E, R, H, 1)(g1, dst, key, norm,
                                           table1.reshape(R * VP, H))

    types2d = nodeTypes.reshape(N, 1)
    h1 = _make_combine1(N, D, H)(types2d, p1[0], p1[1], embpad, root1,
                                 b1.reshape(1, H))

    xr2 = _make_xr2(N, R, H)(h1, W2)
    p2 = _make_edges(N, E, R, H, 2)(g2, dst, norme, xr2.reshape(R * N, H))

    h2 = _make_combine2(N, H)(h1, p2[0], p2[1], root2, b2.reshape(1, H))

    out = _make_pool(N, H)(h2, bs.reshape(N, 1), att_w.reshape(H, 1),
                           lin_w, lin_b.reshape(1, 1))
    return out[:, :1]


# speedup vs baseline: 1.3336x; 1.0037x over previous
import functools

import jax
import jax.numpy as jnp
from jax import lax
from jax.experimental import pallas as pl
from jax.experimental.pallas import tpu as pltpu
from jax.experimental.pallas import tpu_sc as plsc

F32 = jnp.float32
I32 = jnp.int32
VP = 128
B = 256
BN = 1000
NW = 32


def _make_cnt(N, E, R, V):
    KC = 128
    RC = 16
    EW = E // NW
    NCH = EW // KC
    assert EW == NCH * KC + RC
    ZR = 80
    NU = N // ZR
    mesh = plsc.VectorSubcoreMesh(core_axis_name="c", subcore_axis_name="s")

    @functools.partial(
        pl.kernel, mesh=mesh,
        out_type=(jax.ShapeDtypeStruct((2, N, 128), F32),
                  jax.ShapeDtypeStruct((E,), I32)),
        scratch_types=[
            pltpu.VMEM((KC,), I32),
            pltpu.VMEM((KC,), I32),
            pltpu.VMEM((KC,), I32),
            pltpu.VMEM((KC,), I32),
            pltpu.VMEM((KC,), I32),
            pltpu.VMEM((KC,), I32),
            pltpu.VMEM((KC,), I32),
            pltpu.VMEM((RC,), I32),
            pltpu.VMEM((RC,), I32),
            pltpu.VMEM((RC,), I32),
            pltpu.VMEM((RC,), I32),
            pltpu.VMEM((KC, 128), F32),
            pltpu.VMEM((ZR, 128), F32),
            pltpu.VMEM_SHARED((N, 128), F32),
            pltpu.SemaphoreType.DMA,
            pltpu.SemaphoreType.DMA,
        ],
    )
    def k(src_hbm, dst_hbm, rel_hbm, types_hbm, out_hbm, tsrc_hbm,
          sb0, sb1, db0, db1, rb0, rb1, tsrcb, srcb2, dstb2, relb2, tsrcb2,
          ohb, zb, acc, lsem0, lsem1):
        sb_ = (sb0, sb1)
        db_ = (db0, db1)
        rb_ = (rb0, rb1)
        lsem_ = (lsem0, lsem1)
        cid = lax.axis_index("c")
        sid = lax.axis_index("s")
        wid = sid * 2 + cid
        base = wid * EW
        zeros16 = jnp.zeros((16,), F32)
        iota = lax.iota(I32, 16)

        def zrow(i, carry):
            for j in range(8):
                zb[i, pl.ds(j * 16, 16)] = zeros16
            return carry
        lax.fori_loop(0, ZR, zrow, 0)

        def zoh(i, carry):
            for j in range(8):
                ohb[i, pl.ds(j * 16, 16)] = zeros16
            return carry
        lax.fori_loop(0, KC, zoh, 0)
        for t in range(pl.cdiv(NU, 16)):
            u = sid + t * 16

            @pl.when(u < NU)
            def _():
                pltpu.sync_copy(zb, acc.at[pl.ds(u * ZR, ZR)])
        plsc.subcore_barrier()

        def start3(off, bn):
            pltpu.async_copy(src_hbm.at[pl.ds(off, KC)], sb_[bn], lsem_[bn])
            pltpu.async_copy(dst_hbm.at[pl.ds(off, KC)], db_[bn], lsem_[bn])
            pltpu.async_copy(rel_hbm.at[pl.ds(off, KC)], rb_[bn], lsem_[bn])

        def wait3(off, bi):
            pltpu.make_async_copy(src_hbm.at[pl.ds(off, KC)], sb_[bi],
                                  lsem_[bi]).wait()
            pltpu.make_async_copy(dst_hbm.at[pl.ds(off, KC)], db_[bi],
                                  lsem_[bi]).wait()
            pltpu.make_async_copy(rel_hbm.at[pl.ds(off, KC)], rb_[bi],
                                  lsem_[bi]).wait()

        def work(off, bi):
            wait3(off, bi)
            pltpu.sync_copy(types_hbm.at[sb_[bi]], tsrcb)
            pltpu.sync_copy(tsrcb, tsrc_hbm.at[pl.ds(off, KC)])
            rb = rb_[bi]

            def grp(j, carry2):
                r16 = rb[pl.ds(j * 16, 16)]
                for kk in range(16):
                    rv = jnp.full((16,), r16[kk], I32)
                    ohb[j * 16 + kk, pl.ds(0, 16)] = (
                        jnp.where(iota == rv, 1.0, 0.0).astype(F32))
                return carry2
            lax.fori_loop(0, KC // 16, grp, 0)
            pltpu.sync_copy(ohb, acc.at[db_[bi]], add=True)

        start3(base, 0)

        def pair(o, carry):
            for b in range(2):
                i = 2 * o + b
                off = base + i * KC

                @pl.when(i + 1 < NCH)
                def _():
                    start3(base + (i + 1) * KC, 1 - b)
                work(off, b)
            return carry
        lax.fori_loop(0, NCH // 2, pair, 0)

        roff = base + NCH * KC
        pltpu.sync_copy(src_hbm.at[pl.ds(roff, RC)], srcb2)
        pltpu.sync_copy(dst_hbm.at[pl.ds(roff, RC)], dstb2)
        pltpu.sync_copy(rel_hbm.at[pl.ds(roff, RC)], relb2)
        pltpu.sync_copy(types_hbm.at[srcb2], tsrcb2)
        pltpu.sync_copy(tsrcb2, tsrc_hbm.at[pl.ds(roff, RC)])

        def grp2(j, carry2):
            r16 = relb2[pl.ds(j * 16, 16)]
            for kk in range(16):
                rv = jnp.full((16,), r16[kk], I32)
                ohb[j * 16 + kk, pl.ds(0, 16)] = (
                    jnp.where(iota == rv, 1.0, 0.0).astype(F32))
            return carry2
        lax.fori_loop(0, RC // 16, grp2, 0)
        pltpu.sync_copy(ohb.at[pl.ds(0, RC)], acc.at[dstb2], add=True)

        plsc.subcore_barrier()
        for t in range(pl.cdiv(NU, 16)):
            u = sid + t * 16

            @pl.when(u < NU)
            def _():
                pltpu.sync_copy(acc.at[pl.ds(u * ZR, ZR)],
                                out_hbm.at[cid, pl.ds(u * ZR, ZR)])

    return k


def _make_edges(N, E, R, H, layer):
    KC = 128
    RC = 16
    EW = E // NW
    NCH = EW // KC
    assert EW == NCH * KC + RC
    ZR = 80
    NU = N // ZR
    mesh = plsc.VectorSubcoreMesh(core_axis_name="c", subcore_axis_name="s")

    scratch = [
        pltpu.VMEM((KC,), I32),
        pltpu.VMEM((KC,), I32),
        pltpu.VMEM((KC,), I32),
        pltpu.VMEM((KC,), I32),
        pltpu.VMEM((KC,), F32),
        pltpu.VMEM((KC,), F32),
        pltpu.VMEM((RC,), I32),
        pltpu.VMEM((RC,), I32),
        pltpu.VMEM((RC,), F32),
        pltpu.VMEM((KC, H), F32),
        pltpu.VMEM((KC, H), F32),
        pltpu.VMEM((ZR, H), F32),
        pltpu.VMEM_SHARED((N, H), F32),
        pltpu.SemaphoreType.DMA,
        pltpu.SemaphoreType.DMA,
    ]
    if layer == 1:
        scratch += [pltpu.VMEM((KC,), I32), pltpu.VMEM((RC,), I32)]
        out_type = (jax.ShapeDtypeStruct((2, N, H), F32),
                    jax.ShapeDtypeStruct((E,), F32))
    else:
        out_type = jax.ShapeDtypeStruct((2, N, H), F32)

    def body(*args):
        if layer == 1:
            (gidx_hbm, dst_hbm, key_hbm, nflat_hbm, table_hbm,
             out_hbm, norme_hbm, g0, g1, d0, d1, n0, n1, gidxb2, dstb2, nrmb2,
             rows0, rows1, zb, acc, sem0, sem1, keyb, keyb2) = args
        else:
            (gidx_hbm, dst_hbm, norme_hbm, table_hbm,
             out_hbm, g0, g1, d0, d1, n0, n1, gidxb2, dstb2, nrmb2,
             rows0, rows1, zb, acc, sem0, sem1) = args
        gb_ = (g0, g1)
        db_ = (d0, d1)
        nb_ = (n0, n1)
        rows_ = (rows0, rows1)
        sem_ = (sem0, sem1)
        cid = lax.axis_index("c")
        sid = lax.axis_index("s")
        wid = sid * 2 + cid
        base = wid * EW
        zeros16 = jnp.zeros((16,), F32)

        def zrow(i, carry):
            for j in range(H // 16):
                zb[i, pl.ds(j * 16, 16)] = zeros16
            return carry
        lax.fori_loop(0, ZR, zrow, 0)
        for t in range(pl.cdiv(NU, 16)):
            u = sid + t * 16

            @pl.when(u < NU)
            def _():
                pltpu.sync_copy(zb, acc.at[pl.ds(u * ZR, ZR)])
        plsc.subcore_barrier()

        def load_idx(off, bn):
            pltpu.sync_copy(gidx_hbm.at[pl.ds(off, KC)], gb_[bn])
            pltpu.sync_copy(dst_hbm.at[pl.ds(off, KC)], db_[bn])
            if layer == 1:
                pltpu.sync_copy(key_hbm.at[pl.ds(off, KC)], keyb)
                pltpu.sync_copy(nflat_hbm.at[keyb], nb_[bn])
                pltpu.sync_copy(nb_[bn], norme_hbm.at[pl.ds(off, KC)])
            else:
                pltpu.sync_copy(norme_hbm.at[pl.ds(off, KC)], nb_[bn])
            pltpu.async_copy(table_hbm.at[gb_[bn]], rows_[bn], sem_[bn])

        def finish_chunk(off, bi):
            pltpu.make_async_copy(table_hbm.at[gb_[bi]], rows_[bi],
                                  sem_[bi]).wait()
            rows = rows_[bi]
            nb = nb_[bi]

            def scale(jg, carry2):
                nv16 = nb[pl.ds(jg * 16, 16)]
                for kk in range(16):
                    row = jg * 16 + kk
                    nv = jnp.full((16,), nv16[kk], F32)
                    for j in range(H // 16):
                        rows[row, pl.ds(j * 16, 16)] = (
                            rows[row, pl.ds(j * 16, 16)] * nv)
                return carry2
            lax.fori_loop(0, KC // 16, scale, 0)
            pltpu.sync_copy(rows, acc.at[db_[bi]], add=True)

        load_idx(base, 0)

        def pair(o, carry):
            for b in range(2):
                i = 2 * o + b
                load_idx(base + (i + 1) * KC, 1 - b)
                finish_chunk(base + i * KC, b)
            return carry
        lax.fori_loop(0, (NCH - 1) // 2, pair, 0)
        if (NCH - 1) % 2:
            i = NCH - 2
            load_idx(base + (i + 1) * KC, (i + 1) % 2)
            finish_chunk(base + i * KC, i % 2)
        finish_chunk(base + (NCH - 1) * KC, (NCH - 1) % 2)

        roff = base + NCH * KC
        pltpu.sync_copy(gidx_hbm.at[pl.ds(roff, RC)], gidxb2)
        pltpu.sync_copy(dst_hbm.at[pl.ds(roff, RC)], dstb2)
        if layer == 1:
            pltpu.sync_copy(key_hbm.at[pl.ds(roff, RC)], keyb2)
            pltpu.sync_copy(nflat_hbm.at[keyb2], nrmb2)
            pltpu.sync_copy(nrmb2, norme_hbm.at[pl.ds(roff, RC)])
        else:
            pltpu.sync_copy(norme_hbm.at[pl.ds(roff, RC)], nrmb2)
        rslice = rows0.at[pl.ds(0, RC)]
        pltpu.sync_copy(table_hbm.at[gidxb2], rslice)

        def scale2(jg, carry2):
            nv16 = nrmb2[pl.ds(jg * 16, 16)]
            for kk in range(16):
                row = jg * 16 + kk
                nv = jnp.full((16,), nv16[kk], F32)
                for j in range(H // 16):
                    rows0[row, pl.ds(j * 16, 16)] = (
                        rows0[row, pl.ds(j * 16, 16)] * nv)
            return carry2
        lax.fori_loop(0, RC // 16, scale2, 0)
        pltpu.sync_copy(rslice, acc.at[dstb2], add=True)

        plsc.subcore_barrier()
        for t in range(pl.cdiv(NU, 16)):
            u = sid + t * 16

            @pl.when(u < NU)
            def _():
                pltpu.sync_copy(acc.at[pl.ds(u * ZR, ZR)],
                                out_hbm.at[cid, pl.ds(u * ZR, ZR)])

    return functools.partial(
        pl.kernel, mesh=mesh, out_type=out_type, scratch_types=scratch,
    )(body)


def _table1_body(embp_ref, w_ref, out_ref):
    out_ref[0] = jnp.dot(embp_ref[...], w_ref[0], preferred_element_type=F32)


def _make_table1(R, D, H):
    return pl.pallas_call(
        _table1_body,
        grid=(R,),
        in_specs=[
            pl.BlockSpec((VP, D), lambda r: (0, 0)),
            pl.BlockSpec((1, D, H), lambda r: (r, 0, 0)),
        ],
        out_specs=pl.BlockSpec((1, VP, H), lambda r: (r, 0, 0)),
        out_shape=jax.ShapeDtypeStruct((R, VP, H), F32),
    )


def _make_prep(N, E, R):
    rows = E // 128

    def body(src_ref, dst_ref, rel_ref, tsrc_ref, g1_ref, g2_ref, key_ref):
        rel = rel_ref[...]
        g1_ref[...] = rel * VP + tsrc_ref[...]
        g2_ref[...] = rel * N + src_ref[...]
        key_ref[...] = dst_ref[...] * 128 + rel

    spec = pl.BlockSpec((rows, 128), lambda: (0, 0))
    return pl.pallas_call(
        body,
        in_specs=[spec] * 4,
        out_specs=[spec] * 3,
        out_shape=[jax.ShapeDtypeStruct((rows, 128), I32)] * 3,
    )


def _combine1_body(t_ref, p0_ref, p1_ref, embp_ref, root_ref, b_ref, out_ref):
    t = t_ref[...]
    oh = (t == lax.broadcasted_iota(jnp.int32, (1, VP), 1)).astype(F32)
    x = jnp.dot(oh, embp_ref[...], preferred_element_type=F32)
    acc = p0_ref[...] + p1_ref[...] + jnp.dot(x, root_ref[...], preferred_element_type=F32)
    out_ref[...] = jnp.maximum(acc + b_ref[...], 0.0)


def _make_combine1(N, D, H):
    return pl.pallas_call(
        _combine1_body,
        grid=(N // BN,),
        in_specs=[
            pl.BlockSpec((BN, 1), lambda i: (i, 0)),
            pl.BlockSpec((BN, H), lambda i: (i, 0)),
            pl.BlockSpec((BN, H), lambda i: (i, 0)),
            pl.BlockSpec((VP, D), lambda i: (0, 0)),
            pl.BlockSpec((D, H), lambda i: (0, 0)),
            pl.BlockSpec((1, H), lambda i: (0, 0)),
        ],
        out_specs=pl.BlockSpec((BN, H), lambda i: (i, 0)),
        out_shape=jax.ShapeDtypeStruct((N, H), F32),
    )


def _combine2_body(x_ref, p0_ref, p1_ref, root_ref, b_ref, out_ref):
    acc = p0_ref[...] + p1_ref[...] + jnp.dot(
        x_ref[...], root_ref[...], preferred_element_type=F32)
    out_ref[...] = jnp.maximum(acc + b_ref[...], 0.0)


def _make_combine2(N, H):
    return pl.pallas_call(
        _combine2_body,
        grid=(N // BN,),
        in_specs=[
            pl.BlockSpec((BN, H), lambda i: (i, 0)),
            pl.BlockSpec((BN, H), lambda i: (i, 0)),
            pl.BlockSpec((BN, H), lambda i: (i, 0)),
            pl.BlockSpec((H, H), lambda i: (0, 0)),
            pl.BlockSpec((1, H), lambda i: (0, 0)),
        ],
        out_specs=pl.BlockSpec((BN, H), lambda i: (i, 0)),
        out_shape=jax.ShapeDtypeStruct((N, H), F32),
    )


def _xr2_body(h_ref, w_ref, out_ref):
    out_ref[0] = jnp.dot(h_ref[...], w_ref[0], preferred_element_type=F32)


def _make_xr2(N, R, H):
    return pl.pallas_call(
        _xr2_body,
        grid=(R, N // BN),
        in_specs=[
            pl.BlockSpec((BN, H), lambda r, i: (i, 0)),
            pl.BlockSpec((1, H, H), lambda r, i: (r, 0, 0)),
        ],
        out_specs=pl.BlockSpec((1, BN, H), lambda r, i: (r, i, 0)),
        out_shape=jax.ShapeDtypeStruct((R, N, H), F32),
    )


def _pool_body(h_ref, bs_ref, aw_ref, lw_ref, lb_ref, out_ref):
    h = h_ref[...]
    logits = jnp.dot(h, aw_ref[...], preferred_element_type=F32)
    ohb = (bs_ref[...] == lax.broadcasted_iota(jnp.int32, (1, B), 1)).astype(F32)
    neg = jnp.float32(-1e30)
    m = jnp.max(jnp.where(ohb > 0, logits, neg), axis=0, keepdims=True)
    mg = jnp.sum(ohb * m, axis=1, keepdims=True)
    ex = jnp.exp(logits - mg)
    s = jnp.sum(ohb * ex, axis=0, keepdims=True)
    sg = jnp.sum(ohb * s, axis=1, keepdims=True)
    hw = h * (ex / sg)
    ge = lax.dot_general(ohb, hw, (((0,), (0,)), ((), ())),
                         preferred_element_type=F32)
    val = jax.nn.sigmoid(jnp.dot(ge, lw_ref[...], preferred_element_type=F32)
                         + lb_ref[...])
    out_ref[...] = jnp.broadcast_to(val, out_ref.shape)


def _make_pool(N, H):
    return pl.pallas_call(
        _pool_body,
        grid=(1,),
        in_specs=[
            pl.BlockSpec((N, H), lambda i: (0, 0)),
            pl.BlockSpec((N, 1), lambda i: (0, 0)),
            pl.BlockSpec((H, 1), lambda i: (0, 0)),
            pl.BlockSpec((H, 1), lambda i: (0, 0)),
            pl.BlockSpec((1, 1), lambda i: (0, 0)),
        ],
        out_specs=pl.BlockSpec((B, 128), lambda i: (0, 0)),
        out_shape=jax.ShapeDtypeStruct((B, 128), F32),
    )


def _norm_body(c0_ref, c1_ref, out_ref):
    out_ref[...] = 1.0 / jnp.maximum(c0_ref[...] + c1_ref[...], 1.0)


def _make_norm(rows, cols):
    return pl.pallas_call(
        _norm_body,
        grid=(1,),
        in_specs=[pl.BlockSpec((rows, cols), lambda i: (0, 0)),
                  pl.BlockSpec((rows, cols), lambda i: (0, 0))],
        out_specs=pl.BlockSpec((rows, cols), lambda i: (0, 0)),
        out_shape=jax.ShapeDtypeStruct((rows, cols), F32),
    )


def kernel(nodeTypes, edge_index, edge_attr, bs, emb, W1, root1, b1,
           W2, root2, b2, att_w, lin_w, lin_b):
    N, D = nodeTypes.shape[0], emb.shape[1]
    E = edge_attr.shape[0]
    R, H = W1.shape[0], W1.shape[2]
    V = emb.shape[0]

    src = edge_index[0]
    dst = edge_index[1]
    rel = edge_attr

    embpad = jnp.pad(emb, ((0, VP - V), (0, 0)))
    table1 = _make_table1(R, D, H)(embpad, W1)

    cntp, tsrc = _make_cnt(N, E, R, V)(src, dst, rel, nodeTypes)
    norm = _make_norm(N, 128)(cntp[0], cntp[1]).reshape(N * 128)

    e2 = (E // 128, 128)
    g1, g2, key = _make_prep(N, E, R)(src.reshape(e2), dst.reshape(e2),
                                      rel.reshape(e2), tsrc.reshape(e2))
    g1, g2, key = g1.reshape(E), g2.reshape(E), key.reshape(E)

    p1, norme = _make_edges(N, E, R, H, 1)(g1, dst, key, norm,
                                           table1.reshape(R * VP, H))

    types2d = nodeTypes.reshape(N, 1)
    h1 = _make_combine1(N, D, H)(types2d, p1[0], p1[1], embpad, root1,
                                 b1.reshape(1, H))

    xr2 = _make_xr2(N, R, H)(h1, W2)
    p2 = _make_edges(N, E, R, H, 2)(g2, dst, norme, xr2.reshape(R * N, H))

    h2 = _make_combine2(N, H)(h1, p2[0], p2[1], root2, b2.reshape(1, H))

    out = _make_pool(N, H)(h2, bs.reshape(N, 1), att_w.reshape(H, 1),
                           lin_w, lin_b.reshape(1, 1))
    return out[:, :1]


# speedup vs baseline: 1.3508x; 1.0129x over previous
import functools

import jax
import jax.numpy as jnp
from jax import lax
from jax.experimental import pallas as pl
from jax.experimental.pallas import tpu as pltpu
from jax.experimental.pallas import tpu_sc as plsc

F32 = jnp.float32
I32 = jnp.int32
VP = 128
B = 256
BN = 1000
NW = 32


def _make_cnt(N, E, R, V):
    KC = 128
    RC = 16
    EW = E // NW
    NCH = EW // KC
    assert EW == NCH * KC + RC
    ZR = 80
    NU = N // ZR
    mesh = plsc.VectorSubcoreMesh(core_axis_name="c", subcore_axis_name="s")

    @functools.partial(
        pl.kernel, mesh=mesh,
        out_type=(jax.ShapeDtypeStruct((2, N, 128), F32),
                  jax.ShapeDtypeStruct((E,), I32)),
        scratch_types=[
            pltpu.VMEM((KC,), I32),
            pltpu.VMEM((KC,), I32),
            pltpu.VMEM((KC,), I32),
            pltpu.VMEM((KC,), I32),
            pltpu.VMEM((KC,), I32),
            pltpu.VMEM((KC,), I32),
            pltpu.VMEM((KC,), I32),
            pltpu.VMEM((RC,), I32),
            pltpu.VMEM((RC,), I32),
            pltpu.VMEM((RC,), I32),
            pltpu.VMEM((RC,), I32),
            pltpu.VMEM((KC, 128), F32),
            pltpu.VMEM((ZR, 128), F32),
            pltpu.VMEM_SHARED((N, 128), F32),
            pltpu.SemaphoreType.DMA,
            pltpu.SemaphoreType.DMA,
        ],
    )
    def k(src_hbm, dst_hbm, rel_hbm, types_hbm, out_hbm, tsrc_hbm,
          sb0, sb1, db0, db1, rb0, rb1, tsrcb, srcb2, dstb2, relb2, tsrcb2,
          ohb, zb, acc, lsem0, lsem1):
        sb_ = (sb0, sb1)
        db_ = (db0, db1)
        rb_ = (rb0, rb1)
        lsem_ = (lsem0, lsem1)
        cid = lax.axis_index("c")
        sid = lax.axis_index("s")
        wid = sid * 2 + cid
        base = wid * EW
        zeros16 = jnp.zeros((16,), F32)
        iota = lax.iota(I32, 16)

        def zrow(i, carry):
            for j in range(8):
                zb[i, pl.ds(j * 16, 16)] = zeros16
            return carry
        lax.fori_loop(0, ZR, zrow, 0)

        def zoh(i, carry):
            for j in range(8):
                ohb[i, pl.ds(j * 16, 16)] = zeros16
            return carry
        lax.fori_loop(0, KC, zoh, 0)
        for t in range(pl.cdiv(NU, 16)):
            u = sid + t * 16

            @pl.when(u < NU)
            def _():
                pltpu.sync_copy(zb, acc.at[pl.ds(u * ZR, ZR)])
        plsc.subcore_barrier()

        def start3(off, bn):
            pltpu.async_copy(src_hbm.at[pl.ds(off, KC)], sb_[bn], lsem_[bn])
            pltpu.async_copy(dst_hbm.at[pl.ds(off, KC)], db_[bn], lsem_[bn])
            pltpu.async_copy(rel_hbm.at[pl.ds(off, KC)], rb_[bn], lsem_[bn])

        def wait3(off, bi):
            pltpu.make_async_copy(src_hbm.at[pl.ds(off, KC)], sb_[bi],
                                  lsem_[bi]).wait()
            pltpu.make_async_copy(dst_hbm.at[pl.ds(off, KC)], db_[bi],
                                  lsem_[bi]).wait()
            pltpu.make_async_copy(rel_hbm.at[pl.ds(off, KC)], rb_[bi],
                                  lsem_[bi]).wait()

        def work(off, bi):
            wait3(off, bi)
            pltpu.sync_copy(types_hbm.at[sb_[bi]], tsrcb)
            pltpu.sync_copy(tsrcb, tsrc_hbm.at[pl.ds(off, KC)])
            rb = rb_[bi]

            def grp(j, carry2):
                r16 = rb[pl.ds(j * 16, 16)]
                for kk in range(16):
                    rv = jnp.full((16,), r16[kk], I32)
                    ohb[j * 16 + kk, pl.ds(0, 16)] = (
                        jnp.where(iota == rv, 1.0, 0.0).astype(F32))
                return carry2
            lax.fori_loop(0, KC // 16, grp, 0)
            pltpu.sync_copy(ohb, acc.at[db_[bi]], add=True)

        start3(base, 0)

        def pair(o, carry):
            for b in range(2):
                i = 2 * o + b
                off = base + i * KC

                @pl.when(i + 1 < NCH)
                def _():
                    start3(base + (i + 1) * KC, 1 - b)
                work(off, b)
            return carry
        lax.fori_loop(0, NCH // 2, pair, 0)

        roff = base + NCH * KC
        pltpu.sync_copy(src_hbm.at[pl.ds(roff, RC)], srcb2)
        pltpu.sync_copy(dst_hbm.at[pl.ds(roff, RC)], dstb2)
        pltpu.sync_copy(rel_hbm.at[pl.ds(roff, RC)], relb2)
        pltpu.sync_copy(types_hbm.at[srcb2], tsrcb2)
        pltpu.sync_copy(tsrcb2, tsrc_hbm.at[pl.ds(roff, RC)])

        def grp2(j, carry2):
            r16 = relb2[pl.ds(j * 16, 16)]
            for kk in range(16):
                rv = jnp.full((16,), r16[kk], I32)
                ohb[j * 16 + kk, pl.ds(0, 16)] = (
                    jnp.where(iota == rv, 1.0, 0.0).astype(F32))
            return carry2
        lax.fori_loop(0, RC // 16, grp2, 0)
        pltpu.sync_copy(ohb.at[pl.ds(0, RC)], acc.at[dstb2], add=True)

        plsc.subcore_barrier()
        for t in range(pl.cdiv(NU, 16)):
            u = sid + t * 16

            @pl.when(u < NU)
            def _():
                pltpu.sync_copy(acc.at[pl.ds(u * ZR, ZR)],
                                out_hbm.at[cid, pl.ds(u * ZR, ZR)])

    return k


def _make_edges(N, E, R, H, layer):
    KC = 128
    RC = 16
    EW = E // NW
    NCH = EW // KC
    assert EW == NCH * KC + RC
    ZR = 80
    NU = N // ZR
    mesh = plsc.VectorSubcoreMesh(core_axis_name="c", subcore_axis_name="s")

    scratch = [
        pltpu.VMEM((KC,), I32),
        pltpu.VMEM((KC,), I32),
        pltpu.VMEM((KC,), I32),
        pltpu.VMEM((KC,), I32),
        pltpu.VMEM((KC,), F32),
        pltpu.VMEM((KC,), F32),
        pltpu.VMEM((RC,), I32),
        pltpu.VMEM((RC,), I32),
        pltpu.VMEM((RC,), F32),
        pltpu.VMEM((KC, H), F32),
        pltpu.VMEM((KC, H), F32),
        pltpu.VMEM((ZR, H), F32),
        pltpu.VMEM_SHARED((N, H), F32),
        pltpu.SemaphoreType.DMA,
        pltpu.SemaphoreType.DMA,
    ]
    if layer == 1:
        scratch += [pltpu.VMEM((KC,), I32), pltpu.VMEM((RC,), I32)]
        out_type = (jax.ShapeDtypeStruct((2, N, H), F32),
                    jax.ShapeDtypeStruct((E,), F32))
    else:
        out_type = jax.ShapeDtypeStruct((2, N, H), F32)

    def body(*args):
        if layer == 1:
            (gidx_hbm, dst_hbm, key_hbm, nflat_hbm, table_hbm,
             out_hbm, norme_hbm, g0, g1, d0, d1, n0, n1, gidxb2, dstb2, nrmb2,
             rows0, rows1, zb, acc, sem0, sem1, keyb, keyb2) = args
        else:
            (gidx_hbm, dst_hbm, norme_hbm, table_hbm,
             out_hbm, g0, g1, d0, d1, n0, n1, gidxb2, dstb2, nrmb2,
             rows0, rows1, zb, acc, sem0, sem1) = args
        gb_ = (g0, g1)
        db_ = (d0, d1)
        nb_ = (n0, n1)
        rows_ = (rows0, rows1)
        sem_ = (sem0, sem1)
        cid = lax.axis_index("c")
        sid = lax.axis_index("s")
        wid = sid * 2 + cid
        base = wid * EW
        zeros16 = jnp.zeros((16,), F32)

        def zrow(i, carry):
            for j in range(H // 16):
                zb[i, pl.ds(j * 16, 16)] = zeros16
            return carry
        lax.fori_loop(0, ZR, zrow, 0)
        for t in range(pl.cdiv(NU, 16)):
            u = sid + t * 16

            @pl.when(u < NU)
            def _():
                pltpu.sync_copy(zb, acc.at[pl.ds(u * ZR, ZR)])
        plsc.subcore_barrier()

        def load_idx(off, bn):
            pltpu.sync_copy(gidx_hbm.at[pl.ds(off, KC)], gb_[bn])
            pltpu.sync_copy(dst_hbm.at[pl.ds(off, KC)], db_[bn])
            if layer == 1:
                pltpu.sync_copy(key_hbm.at[pl.ds(off, KC)], keyb)
                pltpu.sync_copy(nflat_hbm.at[keyb], nb_[bn])
                pltpu.sync_copy(nb_[bn], norme_hbm.at[pl.ds(off, KC)])
            else:
                pltpu.sync_copy(norme_hbm.at[pl.ds(off, KC)], nb_[bn])
            pltpu.async_copy(table_hbm.at[gb_[bn]], rows_[bn], sem_[bn])

        def finish_chunk(off, bi):
            pltpu.make_async_copy(table_hbm.at[gb_[bi]], rows_[bi],
                                  sem_[bi]).wait()
            rows = rows_[bi]
            nb = nb_[bi]

            def scale(jg, carry2):
                nv16 = nb[pl.ds(jg * 16, 16)]
                for kk in range(16):
                    row = jg * 16 + kk
                    nv = jnp.full((16,), nv16[kk], F32)
                    for j in range(H // 16):
                        rows[row, pl.ds(j * 16, 16)] = (
                            rows[row, pl.ds(j * 16, 16)] * nv)
                return carry2
            lax.fori_loop(0, KC // 16, scale, 0)
            pltpu.sync_copy(rows, acc.at[db_[bi]], add=True)

        load_idx(base, 0)

        def pair(o, carry):
            for b in range(2):
                i = 2 * o + b
                load_idx(base + (i + 1) * KC, 1 - b)
                finish_chunk(base + i * KC, b)
            return carry
        lax.fori_loop(0, (NCH - 1) // 2, pair, 0)
        if (NCH - 1) % 2:
            i = NCH - 2
            load_idx(base + (i + 1) * KC, (i + 1) % 2)
            finish_chunk(base + i * KC, i % 2)
        finish_chunk(base + (NCH - 1) * KC, (NCH - 1) % 2)

        roff = base + NCH * KC
        pltpu.sync_copy(gidx_hbm.at[pl.ds(roff, RC)], gidxb2)
        pltpu.sync_copy(dst_hbm.at[pl.ds(roff, RC)], dstb2)
        if layer == 1:
            pltpu.sync_copy(key_hbm.at[pl.ds(roff, RC)], keyb2)
            pltpu.sync_copy(nflat_hbm.at[keyb2], nrmb2)
            pltpu.sync_copy(nrmb2, norme_hbm.at[pl.ds(roff, RC)])
        else:
            pltpu.sync_copy(norme_hbm.at[pl.ds(roff, RC)], nrmb2)
        rslice = rows0.at[pl.ds(0, RC)]
        pltpu.sync_copy(table_hbm.at[gidxb2], rslice)

        def scale2(jg, carry2):
            nv16 = nrmb2[pl.ds(jg * 16, 16)]
            for kk in range(16):
                row = jg * 16 + kk
                nv = jnp.full((16,), nv16[kk], F32)
                for j in range(H // 16):
                    rows0[row, pl.ds(j * 16, 16)] = (
                        rows0[row, pl.ds(j * 16, 16)] * nv)
            return carry2
        lax.fori_loop(0, RC // 16, scale2, 0)
        pltpu.sync_copy(rslice, acc.at[dstb2], add=True)

        plsc.subcore_barrier()
        for t in range(pl.cdiv(NU, 16)):
            u = sid + t * 16

            @pl.when(u < NU)
            def _():
                pltpu.sync_copy(acc.at[pl.ds(u * ZR, ZR)],
                                out_hbm.at[cid, pl.ds(u * ZR, ZR)])

    return functools.partial(
        pl.kernel, mesh=mesh, out_type=out_type, scratch_types=scratch,
    )(body)


def _table1_body(embp_ref, w_ref, out_ref):
    out_ref[0] = jnp.dot(embp_ref[...], w_ref[0], preferred_element_type=F32)


def _make_table1(R, D, H):
    return pl.pallas_call(
        _table1_body,
        grid=(R,),
        in_specs=[
            pl.BlockSpec((VP, D), lambda r: (0, 0)),
            pl.BlockSpec((1, D, H), lambda r: (r, 0, 0)),
        ],
        out_specs=pl.BlockSpec((1, VP, H), lambda r: (r, 0, 0)),
        out_shape=jax.ShapeDtypeStruct((R, VP, H), F32),
    )


def _make_prep(N, E, R):
    rows = E // 128

    def body(src_ref, dst_ref, rel_ref, tsrc_ref, c0_ref, c1_ref,
             g1_ref, g2_ref, key_ref, norm_ref):
        rel = rel_ref[...]
        g1_ref[...] = rel * VP + tsrc_ref[...]
        g2_ref[...] = rel * N + src_ref[...]
        key_ref[...] = dst_ref[...] * 128 + rel
        norm_ref[...] = 1.0 / jnp.maximum(c0_ref[...] + c1_ref[...], 1.0)

    spec = pl.BlockSpec((rows, 128), lambda: (0, 0))
    cspec = pl.BlockSpec((N, 128), lambda: (0, 0))
    return pl.pallas_call(
        body,
        in_specs=[spec] * 4 + [cspec] * 2,
        out_specs=[spec] * 3 + [cspec],
        out_shape=[jax.ShapeDtypeStruct((rows, 128), I32)] * 3
        + [jax.ShapeDtypeStruct((N, 128), F32)],
    )


def _combine1_body(t_ref, p0_ref, p1_ref, embp_ref, root_ref, b_ref, out_ref):
    t = t_ref[...]
    oh = (t == lax.broadcasted_iota(jnp.int32, (1, VP), 1)).astype(F32)
    x = jnp.dot(oh, embp_ref[...], preferred_element_type=F32)
    acc = p0_ref[...] + p1_ref[...] + jnp.dot(x, root_ref[...], preferred_element_type=F32)
    out_ref[...] = jnp.maximum(acc + b_ref[...], 0.0)


def _make_combine1(N, D, H):
    return pl.pallas_call(
        _combine1_body,
        grid=(N // BN,),
        in_specs=[
            pl.BlockSpec((BN, 1), lambda i: (i, 0)),
            pl.BlockSpec((BN, H), lambda i: (i, 0)),
            pl.BlockSpec((BN, H), lambda i: (i, 0)),
            pl.BlockSpec((VP, D), lambda i: (0, 0)),
            pl.BlockSpec((D, H), lambda i: (0, 0)),
            pl.BlockSpec((1, H), lambda i: (0, 0)),
        ],
        out_specs=pl.BlockSpec((BN, H), lambda i: (i, 0)),
        out_shape=jax.ShapeDtypeStruct((N, H), F32),
    )


def _combine2_body(x_ref, p0_ref, p1_ref, root_ref, b_ref, out_ref):
    acc = p0_ref[...] + p1_ref[...] + jnp.dot(
        x_ref[...], root_ref[...], preferred_element_type=F32)
    out_ref[...] = jnp.maximum(acc + b_ref[...], 0.0)


def _make_combine2(N, H):
    return pl.pallas_call(
        _combine2_body,
        grid=(N // BN,),
        in_specs=[
            pl.BlockSpec((BN, H), lambda i: (i, 0)),
            pl.BlockSpec((BN, H), lambda i: (i, 0)),
            pl.BlockSpec((BN, H), lambda i: (i, 0)),
            pl.BlockSpec((H, H), lambda i: (0, 0)),
            pl.BlockSpec((1, H), lambda i: (0, 0)),
        ],
        out_specs=pl.BlockSpec((BN, H), lambda i: (i, 0)),
        out_shape=jax.ShapeDtypeStruct((N, H), F32),
    )


def _xr2_body(h_ref, w_ref, out_ref):
    out_ref[0] = jnp.dot(h_ref[...], w_ref[0], preferred_element_type=F32)


def _make_xr2(N, R, H):
    return pl.pallas_call(
        _xr2_body,
        grid=(R, N // BN),
        in_specs=[
            pl.BlockSpec((BN, H), lambda r, i: (i, 0)),
            pl.BlockSpec((1, H, H), lambda r, i: (r, 0, 0)),
        ],
        out_specs=pl.BlockSpec((1, BN, H), lambda r, i: (r, i, 0)),
        out_shape=jax.ShapeDtypeStruct((R, N, H), F32),
    )


def _pool_body(h1_ref, p0_ref, p1_ref, root_ref, b_ref,
               bs_ref, aw_ref, lw_ref, lb_ref, out_ref):
    acc = p0_ref[...] + p1_ref[...] + jnp.dot(
        h1_ref[...], root_ref[...], preferred_element_type=F32)
    h = jnp.maximum(acc + b_ref[...], 0.0)
    logits = jnp.dot(h, aw_ref[...], preferred_element_type=F32)
    ohb = (bs_ref[...] == lax.broadcasted_iota(jnp.int32, (1, B), 1)).astype(F32)
    neg = jnp.float32(-1e30)
    m = jnp.max(jnp.where(ohb > 0, logits, neg), axis=0, keepdims=True)
    mg = jnp.sum(ohb * m, axis=1, keepdims=True)
    ex = jnp.exp(logits - mg)
    s = jnp.sum(ohb * ex, axis=0, keepdims=True)
    sg = jnp.sum(ohb * s, axis=1, keepdims=True)
    hw = h * (ex / sg)
    ge = lax.dot_general(ohb, hw, (((0,), (0,)), ((), ())),
                         preferred_element_type=F32)
    val = jax.nn.sigmoid(jnp.dot(ge, lw_ref[...], preferred_element_type=F32)
                         + lb_ref[...])
    out_ref[...] = jnp.broadcast_to(val, out_ref.shape)


def _make_pool(N, H):
    return pl.pallas_call(
        _pool_body,
        grid=(1,),
        in_specs=[
            pl.BlockSpec((N, H), lambda i: (0, 0)),
            pl.BlockSpec((N, H), lambda i: (0, 0)),
            pl.BlockSpec((N, H), lambda i: (0, 0)),
            pl.BlockSpec((H, H), lambda i: (0, 0)),
            pl.BlockSpec((1, H), lambda i: (0, 0)),
            pl.BlockSpec((N, 1), lambda i: (0, 0)),
            pl.BlockSpec((H, 1), lambda i: (0, 0)),
            pl.BlockSpec((H, 1), lambda i: (0, 0)),
            pl.BlockSpec((1, 1), lambda i: (0, 0)),
        ],
        out_specs=pl.BlockSpec((B, 128), lambda i: (0, 0)),
        out_shape=jax.ShapeDtypeStruct((B, 128), F32),
    )


def _norm_body(c0_ref, c1_ref, out_ref):
    out_ref[...] = 1.0 / jnp.maximum(c0_ref[...] + c1_ref[...], 1.0)


def _make_norm(rows, cols):
    return pl.pallas_call(
        _norm_body,
        grid=(1,),
        in_specs=[pl.BlockSpec((rows, cols), lambda i: (0, 0)),
                  pl.BlockSpec((rows, cols), lambda i: (0, 0))],
        out_specs=pl.BlockSpec((rows, cols), lambda i: (0, 0)),
        out_shape=jax.ShapeDtypeStruct((rows, cols), F32),
    )


def kernel(nodeTypes, edge_index, edge_attr, bs, emb, W1, root1, b1,
           W2, root2, b2, att_w, lin_w, lin_b):
    N, D = nodeTypes.shape[0], emb.shape[1]
    E = edge_attr.shape[0]
    R, H = W1.shape[0], W1.shape[2]
    V = emb.shape[0]

    src = edge_index[0]
    dst = edge_index[1]
    rel = edge_attr

    embpad = jnp.pad(emb, ((0, VP - V), (0, 0)))
    table1 = _make_table1(R, D, H)(embpad, W1)

    cntp, tsrc = _make_cnt(N, E, R, V)(src, dst, rel, nodeTypes)

    e2 = (E // 128, 128)
    g1, g2, key, norm = _make_prep(N, E, R)(src.reshape(e2), dst.reshape(e2),
                                            rel.reshape(e2), tsrc.reshape(e2),
                                            cntp[0], cntp[1])
    g1, g2, key = g1.reshape(E), g2.reshape(E), key.reshape(E)
    norm = norm.reshape(N * 128)

    p1, norme = _make_edges(N, E, R, H, 1)(g1, dst, key, norm,
                                           table1.reshape(R * VP, H))

    types2d = nodeTypes.reshape(N, 1)
    h1 = _make_combine1(N, D, H)(types2d, p1[0], p1[1], embpad, root1,
                                 b1.reshape(1, H))

    xr2 = _make_xr2(N, R, H)(h1, W2)
    p2 = _make_edges(N, E, R, H, 2)(g2, dst, norme, xr2.reshape(R * N, H))

    out = _make_pool(N, H)(h1, p2[0], p2[1], root2, b2.reshape(1, H),
                           bs.reshape(N, 1), att_w.reshape(H, 1),
                           lin_w, lin_b.reshape(1, 1))
    return out[:, :1]
